# Initial kernel scaffold; baseline (speedup 1.0000x reference)
import jax
import jax.numpy as jnp
from jax.experimental import pallas as pl


def kernel(nodes, senders, receivers, W, b):
    raise NotImplementedError("write your pallas kernel here")


# speedup vs baseline: 8.2537x; 8.2537x over previous
import functools

import jax
import jax.numpy as jnp
from jax import lax
from jax.experimental import pallas as pl
from jax.experimental.pallas import tpu as pltpu
from jax.experimental.pallas import tpu_sc as plsc

N = 1024
E = 8192
PE = 8
H = 128
LANES = 16
TILES = 32
ROWS_PER_TILE = N // TILES
GROUPS = E // LANES


def _sc_build_a(s_hbm, r_hbm, a_hbm, s_v, r_v, acc_v):
    wid = lax.axis_index("s") * 2 + lax.axis_index("c")
    lo = wid * ROWS_PER_TILE

    pltpu.sync_copy(s_hbm, s_v)
    pltpu.sync_copy(r_hbm, r_v)

    def zero_body(i, carry):
        acc_v[pl.ds(i * LANES, LANES)] = jnp.zeros((LANES,), jnp.float32)
        return carry

    lax.fori_loop(0, ROWS_PER_TILE * N // LANES, zero_body, 0)

    ones = jnp.ones((LANES,), jnp.float32)
    lane = lax.iota(jnp.int32, LANES)

    def edge_body(g, carry):
        rr = r_v[pl.ds(g * LANES, LANES)]
        ss = s_v[pl.ds(g * LANES, LANES)]
        idx = (rr - lo) * N + ss
        inm = (rr >= lo) & (rr < lo + ROWS_PER_TILE)
        for l in range(LANES):
            plsc.addupdate_scatter(acc_v, [idx], ones, mask=inm & (lane == l))
        return carry

    lax.fori_loop(0, GROUPS, edge_body, 0)

    pltpu.sync_copy(acc_v, a_hbm.at[pl.ds(lo * N, ROWS_PER_TILE * N)])


@functools.lru_cache(maxsize=1)
def _build_a():
    return pl.kernel(
        _sc_build_a,
        mesh=plsc.VectorSubcoreMesh(core_axis_name="c", subcore_axis_name="s"),
        out_type=jax.ShapeDtypeStruct((N * N,), jnp.float32),
        scratch_types=[
            pltpu.VMEM((E,), jnp.int32),
            pltpu.VMEM((E,), jnp.int32),
            pltpu.VMEM((ROWS_PER_TILE * N,), jnp.float32),
        ],
        compiler_params=pltpu.CompilerParams(needs_layout_passes=False),
    )


def _tc_body(a_ref, w_ref, b_ref, out_ref, m_ref, p_ref):
    k = pl.program_id(0)
    row = lax.broadcasted_iota(jnp.int32, (N, N), 0)
    col = lax.broadcasted_iota(jnp.int32, (N, N), 1)
    eye = row == col

    @pl.when(k == 0)
    def _():
        a = a_ref[...]
        deg = jnp.sum(a, axis=0)
        inv = 1.0 / jnp.maximum(deg, 1.0)
        m = a * inv[None, :]
        m_ref[...] = m
        p_ref[...] = m

    @pl.when(k > 0)
    def _():
        p_ref[...] = jnp.dot(
            m_ref[...], p_ref[...],
            preferred_element_type=jnp.float32,
            precision=jax.lax.Precision.HIGHEST,
        )

    d = jnp.sum(jnp.where(eye, p_ref[...], 0.0), axis=1)
    term = d[:, None] * w_ref[0]

    @pl.when(k == 0)
    def _():
        out_ref[...] = term + b_ref[...]

    @pl.when(k > 0)
    def _():
        out_ref[...] += term


_rw_chain = pl.pallas_call(
    _tc_body,
    grid=(PE,),
    in_specs=[
        pl.BlockSpec((N, N), lambda k: (0, 0)),
        pl.BlockSpec((1, 1, H), lambda k: (k, 0, 0)),
        pl.BlockSpec((1, H), lambda k: (0, 0)),
    ],
    out_specs=pl.BlockSpec((N, H), lambda k: (0, 0)),
    out_shape=jax.ShapeDtypeStruct((N, H), jnp.float32),
    scratch_shapes=[
        pltpu.VMEM((N, N), jnp.float32),
        pltpu.VMEM((N, N), jnp.float32),
    ],
)


@jax.jit
def kernel(nodes, senders, receivers, W, b):
    del nodes
    s32 = senders.astype(jnp.int32)
    r32 = receivers.astype(jnp.int32)
    a = _build_a()(s32, r32).reshape(N, N)
    return _rw_chain(a, W.reshape(PE, 1, H), b.reshape(1, H))


# speedup vs baseline: 13.0257x; 1.5782x over previous
import functools

import jax
import jax.numpy as jnp
from jax import lax
from jax.experimental import pallas as pl
from jax.experimental.pallas import tpu as pltpu
from jax.experimental.pallas import tpu_sc as plsc

N = 1024
E = 8192
PE = 8
H = 128
LANES = 16
TILES = 32
ROWS_PER_TILE = N // TILES
GROUPS = E // LANES


def _sc_build_a(s_hbm, r_hbm, a_hbm, s_v, r_v, acc_v):
    wid = lax.axis_index("s") * 2 + lax.axis_index("c")
    lo = wid * ROWS_PER_TILE

    pltpu.sync_copy(s_hbm, s_v)
    pltpu.sync_copy(r_hbm, r_v)

    def zero_body(i, carry):
        acc_v[pl.ds(i * LANES, LANES)] = jnp.zeros((LANES,), jnp.float32)
        return carry

    lax.fori_loop(0, ROWS_PER_TILE * N // LANES, zero_body, 0)

    ones = jnp.ones((LANES,), jnp.float32)
    lane = lax.iota(jnp.int32, LANES)

    def edge_body(g, carry):
        rr = r_v[pl.ds(g * LANES, LANES)]
        ss = s_v[pl.ds(g * LANES, LANES)]
        idx = (rr - lo) * N + ss
        inm = (rr >= lo) & (rr < lo + ROWS_PER_TILE)
        for l in range(LANES):
            plsc.addupdate_scatter(acc_v, [idx], ones, mask=inm & (lane == l))
        return carry

    lax.fori_loop(0, GROUPS, edge_body, 0)

    pltpu.sync_copy(acc_v, a_hbm.at[pl.ds(lo * N, ROWS_PER_TILE * N)])


@functools.lru_cache(maxsize=1)
def _build_a():
    return pl.kernel(
        _sc_build_a,
        mesh=plsc.VectorSubcoreMesh(core_axis_name="c", subcore_axis_name="s"),
        out_type=jax.ShapeDtypeStruct((N * N,), jnp.float32),
        scratch_types=[
            pltpu.VMEM((E,), jnp.int32),
            pltpu.VMEM((E,), jnp.int32),
            pltpu.VMEM((ROWS_PER_TILE * N,), jnp.float32),
        ],
        compiler_params=pltpu.CompilerParams(needs_layout_passes=False),
    )


def _tc_body(a_ref, w_ref, b_ref, out_ref):
    a = a_ref[...]
    deg = jnp.sum(a, axis=0)
    m = a * (1.0 / jnp.maximum(deg, 1.0))[None, :]

    def dot(x, y):
        return jnp.dot(x, y, preferred_element_type=jnp.float32,
                       precision=jax.lax.Precision.HIGHEST)

    bm = dot(m, m)
    dm = dot(m, bm)
    cm = dot(bm, bm)
    bt = bm.T
    ct = cm.T

    row = lax.broadcasted_iota(jnp.int32, (N, N), 0)
    col = lax.broadcasted_iota(jnp.int32, (N, N), 1)
    eye = row == col
    zero = jnp.zeros((N, N), jnp.float32)

    diags = [
        jnp.sum(jnp.where(eye, m, zero), axis=1),
        jnp.sum(jnp.where(eye, bm, zero), axis=1),
        jnp.sum(m * bt, axis=1),
        jnp.sum(jnp.where(eye, cm, zero), axis=1),
        jnp.sum(m * ct, axis=1),
        jnp.sum(bm * ct, axis=1),
        jnp.sum(dm * ct, axis=1),
        jnp.sum(cm * ct, axis=1),
    ]
    out = jnp.broadcast_to(b_ref[...], (N, H))
    for k in range(PE):
        out = out + diags[k][:, None] * w_ref[k][None, :]
    out_ref[...] = out


_rw_chain = pl.pallas_call(
    _tc_body,
    out_shape=jax.ShapeDtypeStruct((N, H), jnp.float32),
)


@jax.jit
def kernel(nodes, senders, receivers, W, b):
    del nodes
    s32 = senders.astype(jnp.int32)
    r32 = receivers.astype(jnp.int32)
    a = _build_a()(s32, r32).reshape(N, N)
    return _rw_chain(a, W, b.reshape(1, H))


# speedup vs baseline: 18.0716x; 1.3874x over previous
import functools

import jax
import jax.numpy as jnp
from jax import lax
from jax.experimental import pallas as pl
from jax.experimental.pallas import tpu as pltpu
from jax.experimental.pallas import tpu_sc as plsc

N = 1024
E = 8192
PE = 8
H = 128
LANES = 16
TILES = 32
ROWS_PER_TILE = N // TILES
GROUPS = E // LANES


def _sc_build_a(s_hbm, r_hbm, a_hbm, s_v, r_v, acc_v):
    wid = lax.axis_index("s") * 2 + lax.axis_index("c")
    lo = wid * ROWS_PER_TILE

    pltpu.sync_copy(s_hbm, s_v)
    pltpu.sync_copy(r_hbm, r_v)

    def zero_body(i, carry):
        acc_v[pl.ds(i * LANES, LANES)] = jnp.zeros((LANES,), jnp.float32)
        return carry

    lax.fori_loop(0, ROWS_PER_TILE * N // LANES, zero_body, 0)

    ones = jnp.ones((LANES,), jnp.float32)
    lane = lax.iota(jnp.int32, LANES)

    def edge_body(g, carry):
        rr = r_v[pl.ds(g * LANES, LANES)]
        ss = s_v[pl.ds(g * LANES, LANES)]
        idx = (rr - lo) * N + ss
        inm = (rr >= lo) & (rr < lo + ROWS_PER_TILE)

        @pl.when(jnp.any(inm))
        def _():
            for l in range(LANES):
                plsc.addupdate_scatter(
                    acc_v, [idx], ones, mask=inm & (lane == l))

        return carry

    lax.fori_loop(0, GROUPS, edge_body, 0)

    pltpu.sync_copy(acc_v, a_hbm.at[pl.ds(lo * N, ROWS_PER_TILE * N)])


@functools.lru_cache(maxsize=1)
def _build_a():
    return pl.kernel(
        _sc_build_a,
        mesh=plsc.VectorSubcoreMesh(core_axis_name="c", subcore_axis_name="s"),
        out_type=jax.ShapeDtypeStruct((N * N,), jnp.float32),
        scratch_types=[
            pltpu.VMEM((E,), jnp.int32),
            pltpu.VMEM((E,), jnp.int32),
            pltpu.VMEM((ROWS_PER_TILE * N,), jnp.float32),
        ],
        compiler_params=pltpu.CompilerParams(needs_layout_passes=False),
    )


def _tc_body(a_ref, w_ref, b_ref, out_ref):
    a = a_ref[...]
    deg = jnp.sum(a, axis=0)
    m = a * (1.0 / jnp.maximum(deg, 1.0))[None, :]

    def dot(x, y):
        return jnp.dot(x, y, preferred_element_type=jnp.float32)

    bm = dot(m, m)
    dm = dot(m, bm)
    cm = dot(bm, bm)
    bt = bm.T
    ct = cm.T

    row = lax.broadcasted_iota(jnp.int32, (N, N), 0)
    col = lax.broadcasted_iota(jnp.int32, (N, N), 1)
    eye = row == col
    zero = jnp.zeros((N, N), jnp.float32)

    diags = [
        jnp.sum(jnp.where(eye, m, zero), axis=1),
        jnp.sum(jnp.where(eye, bm, zero), axis=1),
        jnp.sum(m * bt, axis=1),
        jnp.sum(jnp.where(eye, cm, zero), axis=1),
        jnp.sum(m * ct, axis=1),
        jnp.sum(bm * ct, axis=1),
        jnp.sum(dm * ct, axis=1),
        jnp.sum(cm * ct, axis=1),
    ]
    out = jnp.broadcast_to(b_ref[...], (N, H))
    for k in range(PE):
        out = out + diags[k][:, None] * w_ref[k][None, :]
    out_ref[...] = out


_rw_chain = pl.pallas_call(
    _tc_body,
    out_shape=jax.ShapeDtypeStruct((N, H), jnp.float32),
)


@jax.jit
def kernel(nodes, senders, receivers, W, b):
    del nodes
    s32 = senders.astype(jnp.int32)
    r32 = receivers.astype(jnp.int32)
    a = _build_a()(s32, r32).reshape(N, N)
    return _rw_chain(a, W, b.reshape(1, H))


# speedup vs baseline: 22.6365x; 1.2526x over previous
import functools

import jax
import jax.numpy as jnp
from jax import lax
from jax.experimental import pallas as pl
from jax.experimental.pallas import tpu as pltpu
from jax.experimental.pallas import tpu_sc as plsc

N = 1024
E = 8192
PE = 8
H = 128
LANES = 16
TILES = 32
ROWS_PER_TILE = N // TILES
GROUPS = E // LANES


def _sc_build_a(s_hbm, r_hbm, z_hbm, a_hbm, s_v, r_v, acc_v,
                sem_s, sem_r, sem_z):
    wid = lax.axis_index("s") * 2 + lax.axis_index("c")
    lo = wid * ROWS_PER_TILE

    cp_s = pltpu.async_copy(s_hbm, s_v, sem_s)
    cp_r = pltpu.async_copy(r_hbm, r_v, sem_r)
    cp_z = pltpu.async_copy(z_hbm, acc_v, sem_z)
    cp_s.wait()
    cp_r.wait()
    cp_z.wait()

    ones = jnp.ones((LANES,), jnp.float32)
    lane = lax.iota(jnp.int32, LANES)

    def edge_body(g, carry):
        rr = r_v[pl.ds(g * LANES, LANES)]
        ss = s_v[pl.ds(g * LANES, LANES)]
        idx = (rr - lo) * N + ss
        inm = (rr >= lo) & (rr < lo + ROWS_PER_TILE)
        for l in range(LANES):
            plsc.addupdate_scatter(acc_v, [idx], ones, mask=inm & (lane == l))
        return carry

    lax.fori_loop(0, GROUPS, edge_body, 0)

    pltpu.sync_copy(acc_v, a_hbm.at[pl.ds(lo * N, ROWS_PER_TILE * N)])


@functools.lru_cache(maxsize=1)
def _build_a():
    return pl.kernel(
        _sc_build_a,
        mesh=plsc.VectorSubcoreMesh(core_axis_name="c", subcore_axis_name="s"),
        out_type=jax.ShapeDtypeStruct((N * N,), jnp.float32),
        scratch_types=[
            pltpu.VMEM((E,), jnp.int32),
            pltpu.VMEM((E,), jnp.int32),
            pltpu.VMEM((ROWS_PER_TILE * N,), jnp.float32),
            pltpu.SemaphoreType.DMA,
            pltpu.SemaphoreType.DMA,
            pltpu.SemaphoreType.DMA,
        ],
        compiler_params=pltpu.CompilerParams(needs_layout_passes=False),
    )


def _tc_body(a_ref, w_ref, b_ref, out_ref):
    a = a_ref[...]
    deg = jnp.sum(a, axis=0)
    m = a * (1.0 / jnp.maximum(deg, 1.0))[None, :]

    def dot(x, y):
        return jnp.dot(x, y, preferred_element_type=jnp.float32)

    bm = dot(m, m)
    dc = dot(jnp.concatenate([m, bm], 0), bm)
    dm = dc[:N]
    cm = dc[N:]
    ct = cm.T

    row = lax.broadcasted_iota(jnp.int32, (N, N), 0)
    col = lax.broadcasted_iota(jnp.int32, (N, N), 1)
    eyef = jnp.where(row == col, 1.0, 0.0)

    diags = [
        jnp.sum(m * eyef, axis=1),
        jnp.sum(bm * eyef, axis=1),
        jnp.sum(dm * eyef, axis=1),
        jnp.sum(cm * eyef, axis=1),
        jnp.sum(m * ct, axis=1),
        jnp.sum(bm * ct, axis=1),
        jnp.sum(dm * ct, axis=1),
        jnp.sum(cm * ct, axis=1),
    ]
    out = jnp.broadcast_to(b_ref[...], (N, H))
    for k in range(PE):
        out = out + diags[k][:, None] * w_ref[k][None, :]
    out_ref[...] = out


_rw_chain = pl.pallas_call(
    _tc_body,
    out_shape=jax.ShapeDtypeStruct((N, H), jnp.float32),
)


@jax.jit
def kernel(nodes, senders, receivers, W, b):
    del nodes
    s32 = senders.astype(jnp.int32)
    r32 = receivers.astype(jnp.int32)
    zeros = jnp.zeros((ROWS_PER_TILE * N,), jnp.float32)
    a = _build_a()(s32, r32, zeros).reshape(N, N)
    return _rw_chain(a, W, b.reshape(1, H))


# speedup vs baseline: 24.9767x; 1.1034x over previous
import functools

import jax
import jax.numpy as jnp
from jax import lax
from jax.experimental import pallas as pl
from jax.experimental.pallas import tpu as pltpu
from jax.experimental.pallas import tpu_sc as plsc

N = 1024
E = 8192
PE = 8
H = 128
LANES = 16
CORES = 2
SUBCORES = 16
ROWS_PER_TILE = N // SUBCORES
E_PER_CORE = E // CORES
GROUPS = E_PER_CORE // LANES


def _sc_build_a(s_hbm, r_hbm, z_hbm, a_hbm, s_v, r_v, acc_v,
                sem_s, sem_r, sem_z):
    c = lax.axis_index("c")
    lo = lax.axis_index("s") * ROWS_PER_TILE

    cp_s = pltpu.async_copy(s_hbm.at[pl.ds(c * E_PER_CORE, E_PER_CORE)],
                            s_v, sem_s)
    cp_r = pltpu.async_copy(r_hbm.at[pl.ds(c * E_PER_CORE, E_PER_CORE)],
                            r_v, sem_r)
    cp_z = pltpu.async_copy(z_hbm, acc_v, sem_z)
    cp_s.wait()
    cp_r.wait()
    cp_z.wait()

    ones = jnp.ones((LANES,), jnp.float32)
    lane = lax.iota(jnp.int32, LANES)

    def edge_body(g, carry):
        rr = r_v[pl.ds(g * LANES, LANES)]
        ss = s_v[pl.ds(g * LANES, LANES)]
        inm = (rr >= lo) & (rr < lo + ROWS_PER_TILE)
        for l in range(LANES):
            plsc.addupdate_scatter(acc_v, [rr - lo, ss], ones,
                                   mask=inm & (lane == l))
        return carry

    lax.fori_loop(0, GROUPS, edge_body, 0)

    pltpu.sync_copy(acc_v, a_hbm.at[c, pl.ds(lo, ROWS_PER_TILE)])


@functools.lru_cache(maxsize=1)
def _build_a():
    return pl.kernel(
        _sc_build_a,
        mesh=plsc.VectorSubcoreMesh(core_axis_name="c", subcore_axis_name="s"),
        out_type=jax.ShapeDtypeStruct((CORES, N, N), jnp.float32),
        scratch_types=[
            pltpu.VMEM((E_PER_CORE,), jnp.int32),
            pltpu.VMEM((E_PER_CORE,), jnp.int32),
            pltpu.VMEM((ROWS_PER_TILE, N), jnp.float32),
            pltpu.SemaphoreType.DMA,
            pltpu.SemaphoreType.DMA,
            pltpu.SemaphoreType.DMA,
        ],
        compiler_params=pltpu.CompilerParams(needs_layout_passes=False),
    )


def _tc_body(a_ref, w_ref, b_ref, out_ref):
    a = a_ref[0] + a_ref[1]
    deg = jnp.sum(a, axis=0)
    m = a * (1.0 / jnp.maximum(deg, 1.0))[None, :]

    def dot(x, y):
        return jnp.dot(x, y, preferred_element_type=jnp.float32)

    bm = dot(m, m)
    dc = dot(jnp.concatenate([m, bm], 0), bm)
    dm = dc[:N]
    cm = dc[N:]
    ct = cm.T

    row = lax.broadcasted_iota(jnp.int32, (N, N), 0)
    col = lax.broadcasted_iota(jnp.int32, (N, N), 1)
    eyef = jnp.where(row == col, 1.0, 0.0)

    diags = [
        jnp.sum(m * eyef, axis=1),
        jnp.sum(bm * eyef, axis=1),
        jnp.sum(dm * eyef, axis=1),
        jnp.sum(cm * eyef, axis=1),
        jnp.sum(m * ct, axis=1),
        jnp.sum(bm * ct, axis=1),
        jnp.sum(dm * ct, axis=1),
        jnp.sum(cm * ct, axis=1),
    ]
    out = jnp.broadcast_to(b_ref[...], (N, H))
    for k in range(PE):
        out = out + diags[k][:, None] * w_ref[k][None, :]
    out_ref[...] = out


_rw_chain = pl.pallas_call(
    _tc_body,
    out_shape=jax.ShapeDtypeStruct((N, H), jnp.float32),
)


@jax.jit
def kernel(nodes, senders, receivers, W, b):
    del nodes
    s32 = senders.astype(jnp.int32)
    r32 = receivers.astype(jnp.int32)
    zeros = jnp.zeros((ROWS_PER_TILE, N), jnp.float32)
    a = _build_a()(s32, r32, zeros)
    return _rw_chain(a, W, b.reshape(1, H))


# speedup vs baseline: 25.2458x; 1.0108x over previous
import functools

import jax
import jax.numpy as jnp
from jax import lax
from jax.experimental import pallas as pl
from jax.experimental.pallas import tpu as pltpu
from jax.experimental.pallas import tpu_sc as plsc

N = 1024
E = 8192
PE = 8
H = 128
LANES = 16
CORES = 2
SUBCORES = 16
ROWS_PER_TILE = N // SUBCORES
E_PER_CORE = E // CORES
GROUPS = E_PER_CORE // LANES


def _sc_build_a(s_hbm, r_hbm, z_hbm, a_hbm, s_v, r_v, acc_v,
                sem_s, sem_r, sem_z):
    c = lax.axis_index("c")
    lo = lax.axis_index("s") * ROWS_PER_TILE

    cp_s = pltpu.async_copy(s_hbm.at[pl.ds(c * E_PER_CORE, E_PER_CORE)],
                            s_v, sem_s)
    cp_r = pltpu.async_copy(r_hbm.at[pl.ds(c * E_PER_CORE, E_PER_CORE)],
                            r_v, sem_r)
    cp_z = pltpu.async_copy(z_hbm, acc_v, sem_z)
    cp_s.wait()
    cp_r.wait()
    cp_z.wait()

    ones = jnp.ones((LANES,), jnp.float32)
    lane = lax.iota(jnp.int32, LANES)

    def edge_body(g, carry):
        rr = r_v[pl.ds(g * LANES, LANES)]
        ss = s_v[pl.ds(g * LANES, LANES)]
        inm = (rr >= lo) & (rr < lo + ROWS_PER_TILE)
        for l in range(LANES):
            plsc.addupdate_scatter(acc_v, [rr - lo, ss], ones,
                                   mask=inm & (lane == l))
        return carry

    lax.fori_loop(0, GROUPS, edge_body, 0)

    pltpu.sync_copy(acc_v, a_hbm.at[c, pl.ds(lo, ROWS_PER_TILE)])


@functools.lru_cache(maxsize=1)
def _build_a():
    return pl.kernel(
        _sc_build_a,
        mesh=plsc.VectorSubcoreMesh(core_axis_name="c", subcore_axis_name="s"),
        out_type=jax.ShapeDtypeStruct((CORES, N, N), jnp.float32),
        scratch_types=[
            pltpu.VMEM((E_PER_CORE,), jnp.int32),
            pltpu.VMEM((E_PER_CORE,), jnp.int32),
            pltpu.VMEM((ROWS_PER_TILE, N), jnp.float32),
            pltpu.SemaphoreType.DMA,
            pltpu.SemaphoreType.DMA,
            pltpu.SemaphoreType.DMA,
        ],
        compiler_params=pltpu.CompilerParams(needs_layout_passes=False),
    )


def _tc_body(a_ref, w_ref, b_ref, out_ref):
    a = a_ref[0] + a_ref[1]
    deg = jnp.sum(a, axis=0)
    m = a * (1.0 / jnp.maximum(deg, 1.0))[None, :]

    def dot(x, y):
        return jnp.dot(x, y, preferred_element_type=jnp.float32)

    bm = dot(m, m)
    dm = dot(m, bm)
    cm = dot(bm, bm)
    ct = cm.T

    row = lax.broadcasted_iota(jnp.int32, (N, N), 0)
    col = lax.broadcasted_iota(jnp.int32, (N, N), 1)
    eyef = jnp.where(row == col, 1.0, 0.0)

    diags = [
        jnp.sum(m * eyef, axis=1),
        jnp.sum(bm * eyef, axis=1),
        jnp.sum(dm * eyef, axis=1),
        jnp.sum(cm * eyef, axis=1),
        jnp.sum(m * ct, axis=1),
        jnp.sum(bm * ct, axis=1),
        jnp.sum(dm * ct, axis=1),
        jnp.sum(cm * ct, axis=1),
    ]
    out = jnp.broadcast_to(b_ref[...], (N, H))
    for k in range(PE):
        out = out + diags[k][:, None] * w_ref[k][None, :]
    out_ref[...] = out


_rw_chain = pl.pallas_call(
    _tc_body,
    out_shape=jax.ShapeDtypeStruct((N, H), jnp.float32),
)


@jax.jit
def kernel(nodes, senders, receivers, W, b):
    del nodes
    s32 = senders.astype(jnp.int32)
    r32 = receivers.astype(jnp.int32)
    zeros = jnp.zeros((ROWS_PER_TILE, N), jnp.float32)
    a = _build_a()(s32, r32, zeros)
    return _rw_chain(a, W, b.reshape(1, H))
